# baseline (speedup 1.0000x reference)
import functools

import jax
import jax.numpy as jnp
from jax import lax
from jax.experimental import pallas as pl
from jax.experimental.pallas import tpu as pltpu
from jax.experimental.pallas import tpu_sc as plsc

_B, _S, _D = 4, 2048, 1024
_NC, _NS, _L = 2, 16, 16
_NW = _NC * _NS
_S_SC = 256
_S_PER_W = _S_SC // _NW
_R = _S_PER_W
_CH = 1
_W = _R * _D
_G = _CH * _B

_BS = 256

_mesh = plsc.VectorSubcoreMesh(core_axis_name="c", subcore_axis_name="s")


@functools.partial(
    pl.kernel,
    out_type=jax.ShapeDtypeStruct((_B * _S_SC * _D,), jnp.float32),
    mesh=_mesh,
    scratch_types=[
        pltpu.VMEM((2, _W), jnp.float32),
        pltpu.VMEM((2, _W), jnp.float32),
        pltpu.VMEM((_W,), jnp.float32),
        pltpu.SemaphoreType.DMA((2,)),
        pltpu.SemaphoreType.DMA((2,)),
        pltpu.SemaphoreType.DMA,
    ],
)
def _sc_add(seq_hbm, pos_hbm, out_hbm, in_v, out_v, pos_v, sem_in, sem_out, sem_pos):
    wid = lax.axis_index("s") * _NC + lax.axis_index("c")
    s_base = wid * _S_PER_W

    in_descs, out_descs = {}, {}

    def start_in(g):
        off = ((g % _B) * _S + s_base) * _D
        in_descs[g] = pltpu.async_copy(
            seq_hbm.at[pl.ds(off, _W)], in_v.at[g % 2], sem_in.at[g % 2])

    def start_out(g):
        off = ((g % _B) * _S_SC + s_base) * _D
        out_descs[g] = pltpu.async_copy(
            out_v.at[g % 2], out_hbm.at[pl.ds(off, _W)], sem_out.at[g % 2])

    pos_desc = pltpu.async_copy(
        pos_hbm.at[pl.ds(s_base * _D, _W)], pos_v, sem_pos)
    start_in(0)
    pos_desc.wait()
    for g in range(_G):
        if g + 1 < _G:
            start_in(g + 1)
        in_descs[g].wait()
        if g >= 2:
            out_descs[g - 2].wait()
        src, dst = in_v.at[g % 2], out_v.at[g % 2]

        @plsc.parallel_loop(0, _W, step=_L, unroll=8)
        def _add(i):
            sl = pl.ds(i, _L)
            dst[sl] = src[sl] + pos_v[sl]

        start_out(g)
    out_descs[_G - 2].wait()
    out_descs[_G - 1].wait()


def _tc_add_kernel(seq_ref, pos_ref, out_ref):
    out_ref[...] = seq_ref[...] + pos_ref[...][None, :, :]


def _tc_call(seq_emb, pos_table):
    n_blocks = (_S - _S_SC) // _BS
    off_blocks = _S_SC // _BS
    return pl.pallas_call(
        _tc_add_kernel,
        grid=(n_blocks,),
        in_specs=[
            pl.BlockSpec((_B, _BS, _D), lambda i: (0, i + off_blocks, 0)),
            pl.BlockSpec((_BS, _D), lambda i: (i + off_blocks, 0)),
        ],
        out_specs=pl.BlockSpec((_B, _BS, _D), lambda i: (0, i, 0)),
        out_shape=jax.ShapeDtypeStruct((_B, _S - _S_SC, _D), seq_emb.dtype),
    )(seq_emb, pos_table)


def kernel(seq_emb, pos_table):
    batch, seq_len, dim = seq_emb.shape
    sc_out = _sc_add(seq_emb.reshape(-1), pos_table.reshape(-1))
    tc_out = _tc_call(seq_emb, pos_table)
    return jnp.concatenate(
        [sc_out.reshape(batch, _S_SC, dim), tc_out], axis=1)
